# Initial kernel scaffold; baseline (speedup 1.0000x reference)
import jax
import jax.numpy as jnp
from jax.experimental import pallas as pl


def kernel(x, edge_index, W, b, gamma, beta, mix_weight, prelu_a):
    raise NotImplementedError("write your pallas kernel here")


# speedup vs baseline: 11.5373x; 11.5373x over previous
import functools

import jax
import jax.numpy as jnp
from jax import lax
from jax.experimental import pallas as pl
from jax.experimental.pallas import tpu as pltpu
from jax.experimental.pallas import tpu_sc as plsc

_NCORE = 2
_NSUB = 16
_LANES = 16


def _chunk_size(per_tile):
    for c in (128, 112, 96, 80, 64, 48, 32, 16):
        if per_tile % c == 0:
            return c
    raise ValueError(f"no chunk size for per-tile edge count {per_tile}")


def _pad_nodes(N):
    return ((N + _NSUB * _LANES - 1) // (_NSUB * _LANES)) * (_NSUB * _LANES)


def _fill_iota(idx_ref, start, cnt):
    assert cnt % _LANES == 0
    for j in range(cnt // _LANES):
        idx_ref[pl.ds(j * _LANES, _LANES)] = (
            lax.iota(jnp.int32, _LANES) + (start + j * _LANES))


def _zero_rows(rows_ref, cnt, D):
    def body(i, carry):
        for j in range(D // _LANES):
            rows_ref[i, pl.ds(j * _LANES, _LANES)] = jnp.zeros((_LANES,), jnp.float32)
        return carry

    lax.fori_loop(0, cnt, body, 0)


@functools.lru_cache(maxsize=None)
def _deg_kernel(N, E, D):
    NP = _pad_nodes(N)
    per_tile = E // (_NCORE * _NSUB)
    assert per_tile * _NCORE * _NSUB == E
    C = _chunk_size(per_tile)
    nchunks = per_tile // C
    rows_pt = NP // _NSUB
    assert rows_pt % C == 0
    mesh = plsc.VectorSubcoreMesh(core_axis_name="c", subcore_axis_name="s")

    @functools.partial(
        pl.kernel,
        out_type=jax.ShapeDtypeStruct((_NCORE * NP, D), jnp.float32),
        mesh=mesh,
        scratch_types=[
            pltpu.VMEM((C,), jnp.int32),
            pltpu.VMEM((C, D), jnp.float32),
            pltpu.VMEM_SHARED((NP, D), jnp.float32),
        ],
    )
    def deg(dst_hbm, out_hbm, idx_d, rows, acc):
        cid = lax.axis_index("c")
        sid = lax.axis_index("s")
        row0 = sid * rows_pt

        _zero_rows(rows, C, D)
        for z in range(rows_pt // C):
            _fill_iota(idx_d, row0 + z * C, C)
            pltpu.sync_copy(rows, acc.at[idx_d])
        plsc.subcore_barrier()

        def fill_ones(i, carry):
            for j in range(D // _LANES):
                rows[i, pl.ds(j * _LANES, _LANES)] = jnp.ones((_LANES,), jnp.float32)
            return carry

        lax.fori_loop(0, C, fill_ones, 0)

        base = (sid * _NCORE + cid) * per_tile

        def body(k, carry):
            off = base + k * C
            pltpu.sync_copy(dst_hbm.at[pl.ds(off, C)], idx_d)
            pltpu.sync_copy(rows, acc.at[idx_d], add=True)
            return carry

        lax.fori_loop(0, nchunks, body, 0)
        plsc.subcore_barrier()

        for z in range(rows_pt // C):
            _fill_iota(idx_d, row0 + z * C, C)
            pltpu.sync_copy(acc.at[idx_d], rows)
            pltpu.sync_copy(rows, out_hbm.at[pl.ds(cid * NP + row0 + z * C, C)])

    return deg


@functools.lru_cache(maxsize=None)
def _spmm_kernel(N, E, D):
    NP = _pad_nodes(N)
    per_tile = E // (_NCORE * _NSUB)
    assert per_tile * _NCORE * _NSUB == E
    C = _chunk_size(per_tile)
    nchunks = per_tile // C
    rows_pt = NP // _NSUB
    assert rows_pt % C == 0
    mesh = plsc.VectorSubcoreMesh(core_axis_name="c", subcore_axis_name="s")

    @functools.partial(
        pl.kernel,
        out_type=jax.ShapeDtypeStruct((_NCORE * NP, D), jnp.float32),
        mesh=mesh,
        scratch_types=[
            pltpu.VMEM((C,), jnp.int32),
            pltpu.VMEM((C,), jnp.int32),
            pltpu.VMEM((C, D), jnp.float32),
            pltpu.VMEM_SHARED((NP, D), jnp.float32),
            pltpu.SemaphoreType.DMA,
        ],
    )
    def spmm(src_hbm, dst_hbm, tbl_hbm, out_hbm, idx_s, idx_d, rows, acc, sem):
        cid = lax.axis_index("c")
        sid = lax.axis_index("s")
        row0 = sid * rows_pt

        _zero_rows(rows, C, D)
        for z in range(rows_pt // C):
            _fill_iota(idx_d, row0 + z * C, C)
            pltpu.sync_copy(rows, acc.at[idx_d])
        plsc.subcore_barrier()

        base = (sid * _NCORE + cid) * per_tile

        def body(k, carry):
            off = base + k * C
            pltpu.sync_copy(src_hbm.at[pl.ds(off, C)], idx_s)
            pltpu.sync_copy(dst_hbm.at[pl.ds(off, C)], idx_d)
            pltpu.async_copy(tbl_hbm.at[idx_s], rows, sem).wait()
            pltpu.sync_copy(rows, acc.at[idx_d], add=True)
            return carry

        lax.fori_loop(0, nchunks, body, 0)
        plsc.subcore_barrier()

        for z in range(rows_pt // C):
            _fill_iota(idx_d, row0 + z * C, C)
            pltpu.sync_copy(acc.at[idx_d], rows)
            pltpu.sync_copy(rows, out_hbm.at[pl.ds(cid * NP + row0 + z * C, C)])

    return spmm


def _bn(z, g, be):
    mu = jnp.mean(z, axis=0, keepdims=True)
    var = jnp.mean((z - mu) ** 2, axis=0, keepdims=True)
    return (z - mu) * lax.rsqrt(var + 1e-5) * g + be


def _prelu(z, a):
    return jnp.where(z >= 0.0, z, a * z)


@functools.lru_cache(maxsize=None)
def _pre_kernel(N, D):
    NP = _pad_nodes(N)

    def body(degp, x, xs, dinv):
        deg = degp[0:N, 0:1] + degp[NP:NP + N, 0:1] + 1.0
        dv = lax.rsqrt(deg)
        dinv[...] = dv
        xs[0:N, :] = x[...] * dv
        if NP > N:
            xs[N:NP, :] = jnp.zeros((NP - N, D), jnp.float32)

    return pl.pallas_call(
        body,
        out_shape=(
            jax.ShapeDtypeStruct((NP, D), jnp.float32),
            jax.ShapeDtypeStruct((N, 1), jnp.float32),
        ),
    )


@functools.lru_cache(maxsize=None)
def _mid_kernel(N, D):
    NP = _pad_nodes(N)

    def body(sx, xs, dinv, bv, gv, bev, av, w1, w2, out):
        dv = dinv[...]
        st = sx[0:N, :] + sx[NP:NP + N, :] + xs[0:N, :]
        conv = jnp.dot(st, w1[...], preferred_element_type=jnp.float32) * dv + bv[...]
        z = _prelu(_bn(conv, gv[...], bev[...]), av[0, 0])
        out[0:N, :] = jnp.dot(z, w2[...], preferred_element_type=jnp.float32) * dv
        if NP > N:
            out[N:NP, :] = jnp.zeros((NP - N, D), jnp.float32)

    return pl.pallas_call(
        body, out_shape=jax.ShapeDtypeStruct((NP, D), jnp.float32))


@functools.lru_cache(maxsize=None)
def _last_kernel(N, D):
    NP = _pad_nodes(N)

    def body(p, hs, dinv, bv, gv, bev, av, out):
        conv = (p[0:N, :] + p[NP:NP + N, :] + hs[0:N, :]) * dinv[...] + bv[...]
        out[...] = _prelu(_bn(conv, gv[...], bev[...]), av[0, 0])

    return pl.pallas_call(
        body, out_shape=jax.ShapeDtypeStruct((N, D), jnp.float32))


@functools.lru_cache(maxsize=None)
def _mix_kernel(N, D):
    def body(za, zb, mixw, out):
        m = jax.nn.softmax(mixw[...], axis=-1)
        out[...] = m[:, 0:1] * za[...] + m[:, 1:2] * zb[...]

    return pl.pallas_call(
        body, out_shape=jax.ShapeDtypeStruct((N, D), jnp.float32))


def kernel(x, edge_index, W, b, gamma, beta, mix_weight, prelu_a):
    N, D = x.shape
    E = edge_index.shape[1]
    src = edge_index[0].astype(jnp.int32)
    dst = edge_index[1].astype(jnp.int32)
    row = lambda v: v.reshape(1, -1)
    pa2 = prelu_a.reshape(1, 1)

    degp = _deg_kernel(N, E, D)(dst)
    xs, dinv = _pre_kernel(N, D)(degp, x)
    sx = _spmm_kernel(N, E, D)(src, dst, xs)
    hs_a = _mid_kernel(N, D)(sx, xs, dinv, row(b[0]), row(gamma[0]),
                             row(beta[0]), pa2, W[0], W[1])
    hs_b = _mid_kernel(N, D)(sx, xs, dinv, row(b[2]), row(gamma[2]),
                             row(beta[2]), pa2, W[2], W[3])
    p_a = _spmm_kernel(N, E, D)(src, dst, hs_a)
    p_b = _spmm_kernel(N, E, D)(src, dst, hs_b)
    za = _last_kernel(N, D)(p_a, hs_a, dinv, row(b[1]), row(gamma[1]),
                            row(beta[1]), pa2)
    zb = _last_kernel(N, D)(p_b, hs_b, dinv, row(b[3]), row(gamma[3]),
                            row(beta[3]), pa2)
    return _mix_kernel(N, D)(za, zb, mix_weight)


# speedup vs baseline: 19.4432x; 1.6852x over previous
import functools

import jax
import jax.numpy as jnp
from jax import lax
from jax.experimental import pallas as pl
from jax.experimental.pallas import tpu as pltpu
from jax.experimental.pallas import tpu_sc as plsc

_NCORE = 2
_NSUB = 16
_LANES = 16


def _chunk_size(per_tile):
    for c in (128, 112, 96, 80, 64, 48, 32, 16):
        if per_tile % c == 0:
            return c
    raise ValueError(f"no chunk size for per-tile edge count {per_tile}")


def _pad_nodes(N):
    return ((N + _NSUB * _LANES - 1) // (_NSUB * _LANES)) * (_NSUB * _LANES)


def _fill_iota(idx_ref, start, cnt):
    assert cnt % _LANES == 0
    for j in range(cnt // _LANES):
        idx_ref[pl.ds(j * _LANES, _LANES)] = (
            lax.iota(jnp.int32, _LANES) + (start + j * _LANES))


def _zero_rows(rows_ref, cnt, D):
    def body(i, carry):
        for j in range(D // _LANES):
            rows_ref[i, pl.ds(j * _LANES, _LANES)] = jnp.zeros((_LANES,), jnp.float32)
        return carry

    lax.fori_loop(0, cnt, body, 0)


@functools.lru_cache(maxsize=None)
def _deg_kernel(N, E, D):
    NP = _pad_nodes(N)
    per_tile = E // (_NCORE * _NSUB)
    assert per_tile * _NCORE * _NSUB == E
    C = _chunk_size(per_tile)
    nchunks = per_tile // C
    rows_pt = NP // _NSUB
    assert rows_pt % C == 0
    mesh = plsc.VectorSubcoreMesh(core_axis_name="c", subcore_axis_name="s")

    @functools.partial(
        pl.kernel,
        out_type=jax.ShapeDtypeStruct((_NCORE * NP, D), jnp.float32),
        mesh=mesh,
        scratch_types=[
            pltpu.VMEM((C,), jnp.int32),
            pltpu.VMEM((C, D), jnp.float32),
            pltpu.VMEM_SHARED((NP, D), jnp.float32),
        ],
    )
    def deg(dst_hbm, out_hbm, idx_d, rows, acc):
        cid = lax.axis_index("c")
        sid = lax.axis_index("s")
        row0 = sid * rows_pt

        _zero_rows(rows, C, D)
        for z in range(rows_pt // C):
            _fill_iota(idx_d, row0 + z * C, C)
            pltpu.sync_copy(rows, acc.at[idx_d])
        plsc.subcore_barrier()

        def fill_ones(i, carry):
            for j in range(D // _LANES):
                rows[i, pl.ds(j * _LANES, _LANES)] = jnp.ones((_LANES,), jnp.float32)
            return carry

        lax.fori_loop(0, C, fill_ones, 0)

        base = (sid * _NCORE + cid) * per_tile

        def body(k, carry):
            off = base + k * C
            pltpu.sync_copy(dst_hbm.at[pl.ds(off, C)], idx_d)
            pltpu.sync_copy(rows, acc.at[idx_d], add=True)
            return carry

        lax.fori_loop(0, nchunks, body, 0)
        plsc.subcore_barrier()

        for z in range(rows_pt // C):
            _fill_iota(idx_d, row0 + z * C, C)
            pltpu.sync_copy(acc.at[idx_d], rows)
            pltpu.sync_copy(rows, out_hbm.at[pl.ds(cid * NP + row0 + z * C, C)])

    return deg


@functools.lru_cache(maxsize=None)
def _spmm_kernel(N, E, D):
    NP = _pad_nodes(N)
    per_tile = E // (_NCORE * _NSUB)
    assert per_tile * _NCORE * _NSUB == E
    C = _chunk_size(per_tile)
    nchunks = per_tile // C
    assert nchunks >= 3
    rows_pt = NP // _NSUB
    assert rows_pt % C == 0
    mesh = plsc.VectorSubcoreMesh(core_axis_name="c", subcore_axis_name="s")

    @functools.partial(
        pl.kernel,
        out_type=jax.ShapeDtypeStruct((_NCORE * NP, D), jnp.float32),
        mesh=mesh,
        scratch_types=[
            pltpu.VMEM((C,), jnp.int32),
            pltpu.VMEM((C,), jnp.int32),
            pltpu.VMEM((C,), jnp.int32),
            pltpu.VMEM((C,), jnp.int32),
            pltpu.VMEM((C, D), jnp.float32),
            pltpu.VMEM((C, D), jnp.float32),
            pltpu.VMEM_SHARED((NP, D), jnp.float32),
            pltpu.SemaphoreType.DMA,
            pltpu.SemaphoreType.DMA,
            pltpu.SemaphoreType.DMA,
            pltpu.SemaphoreType.DMA,
            pltpu.SemaphoreType.DMA,
            pltpu.SemaphoreType.DMA,
        ],
    )
    def spmm(src_hbm, dst_hbm, tbl_hbm, out_hbm, is0, is1, id0, id1, r0, r1,
             acc, sg0, sg1, sis0, sis1, sid0, sid1):
        IS, ID, R = (is0, is1), (id0, id1), (r0, r1)
        SG, SIS, SID = (sg0, sg1), (sis0, sis1), (sid0, sid1)
        cid = lax.axis_index("c")
        sid = lax.axis_index("s")
        row0 = sid * rows_pt

        _zero_rows(r0, C, D)
        for z in range(rows_pt // C):
            _fill_iota(id0, row0 + z * C, C)
            pltpu.sync_copy(r0, acc.at[id0])
        plsc.subcore_barrier()

        ebase = (sid * _NCORE + cid) * per_tile

        def start_idx(k, p):
            off = ebase + k * C
            pltpu.async_copy(src_hbm.at[pl.ds(off, C)], IS[p], SIS[p])
            pltpu.async_copy(dst_hbm.at[pl.ds(off, C)], ID[p], SID[p])

        def wait_idx(p):
            pltpu.make_async_copy(src_hbm.at[pl.ds(0, C)], IS[p], SIS[p]).wait()
            pltpu.make_async_copy(dst_hbm.at[pl.ds(0, C)], ID[p], SID[p]).wait()

        def start_gather(p):
            pltpu.async_copy(tbl_hbm.at[IS[p]], R[p], SG[p])

        def wait_gather(p):
            pltpu.make_async_copy(tbl_hbm.at[IS[p]], R[p], SG[p]).wait()

        def chunk(k, p, next_idx, next_gather):
            wait_gather(p)
            if next_gather:
                q = 1 - p
                wait_idx(q)
                start_gather(q)
            pltpu.sync_copy(R[p], acc.at[ID[p]], add=True)
            if next_idx is not None:
                start_idx(next_idx, p)

        start_idx(0, 0)
        start_idx(1, 1)
        wait_idx(0)
        start_gather(0)

        full_pairs = (nchunks - 2) // 2

        def body(i, carry):
            k = 2 * i
            chunk(k, 0, k + 2, True)
            chunk(k + 1, 1, k + 3, True)
            return carry

        lax.fori_loop(0, full_pairs, body, 0)
        for k in range(2 * full_pairs, nchunks):
            chunk(k, k % 2, k + 2 if k + 2 < nchunks else None, k + 1 < nchunks)
        plsc.subcore_barrier()

        for z in range(rows_pt // C):
            _fill_iota(id0, row0 + z * C, C)
            pltpu.sync_copy(acc.at[id0], r0)
            pltpu.sync_copy(r0, out_hbm.at[pl.ds(cid * NP + row0 + z * C, C)])

    return spmm


def _bn(z, g, be):
    mu = jnp.mean(z, axis=0, keepdims=True)
    var = jnp.mean((z - mu) ** 2, axis=0, keepdims=True)
    return (z - mu) * lax.rsqrt(var + 1e-5) * g + be


def _prelu(z, a):
    return jnp.where(z >= 0.0, z, a * z)


@functools.lru_cache(maxsize=None)
def _pre_kernel(N, D):
    NP = _pad_nodes(N)

    def body(degp, x, xs, dinv):
        deg = degp[0:N, 0:1] + degp[NP:NP + N, 0:1] + 1.0
        dv = lax.rsqrt(deg)
        dinv[...] = dv
        xs[0:N, :] = x[...] * dv
        if NP > N:
            xs[N:NP, :] = jnp.zeros((NP - N, D), jnp.float32)

    return pl.pallas_call(
        body,
        out_shape=(
            jax.ShapeDtypeStruct((NP, D), jnp.float32),
            jax.ShapeDtypeStruct((N, 1), jnp.float32),
        ),
    )


@functools.lru_cache(maxsize=None)
def _mid_kernel(N, D):
    NP = _pad_nodes(N)

    def body(sx, xs, dinv, bv, gv, bev, av, w1, w2, out):
        dv = dinv[...]
        st = sx[0:N, :] + sx[NP:NP + N, :] + xs[0:N, :]
        conv = jnp.dot(st, w1[...], preferred_element_type=jnp.float32) * dv + bv[...]
        z = _prelu(_bn(conv, gv[...], bev[...]), av[0, 0])
        out[0:N, :] = jnp.dot(z, w2[...], preferred_element_type=jnp.float32) * dv
        if NP > N:
            out[N:NP, :] = jnp.zeros((NP - N, D), jnp.float32)

    return pl.pallas_call(
        body, out_shape=jax.ShapeDtypeStruct((NP, D), jnp.float32))


@functools.lru_cache(maxsize=None)
def _last_kernel(N, D):
    NP = _pad_nodes(N)

    def body(p, hs, dinv, bv, gv, bev, av, out):
        conv = (p[0:N, :] + p[NP:NP + N, :] + hs[0:N, :]) * dinv[...] + bv[...]
        out[...] = _prelu(_bn(conv, gv[...], bev[...]), av[0, 0])

    return pl.pallas_call(
        body, out_shape=jax.ShapeDtypeStruct((N, D), jnp.float32))


@functools.lru_cache(maxsize=None)
def _mix_kernel(N, D):
    def body(za, zb, mixw, out):
        m = jax.nn.softmax(mixw[...], axis=-1)
        out[...] = m[:, 0:1] * za[...] + m[:, 1:2] * zb[...]

    return pl.pallas_call(
        body, out_shape=jax.ShapeDtypeStruct((N, D), jnp.float32))


def kernel(x, edge_index, W, b, gamma, beta, mix_weight, prelu_a):
    N, D = x.shape
    E = edge_index.shape[1]
    src = edge_index[0].astype(jnp.int32)
    dst = edge_index[1].astype(jnp.int32)
    row = lambda v: v.reshape(1, -1)
    pa2 = prelu_a.reshape(1, 1)

    degp = _deg_kernel(N, E, D)(dst)
    xs, dinv = _pre_kernel(N, D)(degp, x)
    sx = _spmm_kernel(N, E, D)(src, dst, xs)
    hs_a = _mid_kernel(N, D)(sx, xs, dinv, row(b[0]), row(gamma[0]),
                             row(beta[0]), pa2, W[0], W[1])
    hs_b = _mid_kernel(N, D)(sx, xs, dinv, row(b[2]), row(gamma[2]),
                             row(beta[2]), pa2, W[2], W[3])
    p_a = _spmm_kernel(N, E, D)(src, dst, hs_a)
    p_b = _spmm_kernel(N, E, D)(src, dst, hs_b)
    za = _last_kernel(N, D)(p_a, hs_a, dinv, row(b[1]), row(gamma[1]),
                            row(beta[1]), pa2)
    zb = _last_kernel(N, D)(p_b, hs_b, dinv, row(b[3]), row(gamma[3]),
                            row(beta[3]), pa2)
    return _mix_kernel(N, D)(za, zb, mix_weight)


# speedup vs baseline: 21.4128x; 1.1013x over previous
import functools

import jax
import jax.numpy as jnp
from jax import lax
from jax.experimental import pallas as pl
from jax.experimental.pallas import tpu as pltpu
from jax.experimental.pallas import tpu_sc as plsc

_NCORE = 2
_NSUB = 16
_LANES = 16


def _chunk_size(per_tile):
    for c in (128, 112, 96, 80, 64, 48, 32, 16):
        if per_tile % c == 0:
            return c
    raise ValueError(f"no chunk size for per-tile edge count {per_tile}")


def _pad_nodes(N):
    return ((N + _NSUB * _LANES - 1) // (_NSUB * _LANES)) * (_NSUB * _LANES)


def _fill_iota(idx_ref, start, cnt):
    assert cnt % _LANES == 0
    for j in range(cnt // _LANES):
        idx_ref[pl.ds(j * _LANES, _LANES)] = (
            lax.iota(jnp.int32, _LANES) + (start + j * _LANES))


def _zero_rows(rows_ref, cnt, D):
    def body(i, carry):
        for j in range(D // _LANES):
            rows_ref[i, pl.ds(j * _LANES, _LANES)] = jnp.zeros((_LANES,), jnp.float32)
        return carry

    lax.fori_loop(0, cnt, body, 0)


@functools.lru_cache(maxsize=None)
def _deg_kernel(N, E, D):
    NP = _pad_nodes(N)
    per_tile = E // (_NCORE * _NSUB)
    assert per_tile * _NCORE * _NSUB == E
    C = _chunk_size(per_tile)
    nchunks = per_tile // C
    rows_pt = NP // _NSUB
    assert rows_pt % C == 0
    mesh = plsc.VectorSubcoreMesh(core_axis_name="c", subcore_axis_name="s")

    NB = 4

    @functools.partial(
        pl.kernel,
        out_type=jax.ShapeDtypeStruct((_NCORE * NP, D), jnp.float32),
        mesh=mesh,
        scratch_types=(
            [pltpu.VMEM((C,), jnp.int32)] * NB
            + [pltpu.VMEM((C, D), jnp.float32)] * 2
            + [pltpu.VMEM_SHARED((NP, D), jnp.float32)]
            + [pltpu.SemaphoreType.DMA] * (2 * NB + 2)
        ),
    )
    def deg(dst_hbm, out_hbm, *refs):
        ID = refs[0:NB]
        R = refs[NB:NB + 2]
        acc = refs[NB + 2]
        SS = refs[NB + 3:NB + 3 + NB]
        SI = refs[NB + 3 + NB:NB + 3 + 2 * NB]
        SG = refs[NB + 3 + 2 * NB:NB + 3 + 2 * NB + 2]
        cid = lax.axis_index("c")
        sid = lax.axis_index("s")
        row0 = sid * rows_pt

        def wait_scatter(p):
            pltpu.make_async_copy(R[0], acc.at[ID[p]], SS[p]).wait()

        _zero_rows(R[0], C, D)
        nzc = rows_pt // C
        for z in range(nzc):
            if z >= NB:
                wait_scatter(z % NB)
            _fill_iota(ID[z % NB], row0 + z * C, C)
            pltpu.async_copy(R[0], acc.at[ID[z % NB]], SS[z % NB])
        for z in range(max(nzc - NB, 0), nzc):
            wait_scatter(z % NB)

        def fill_ones(i, carry):
            for j in range(D // _LANES):
                R[0][i, pl.ds(j * _LANES, _LANES)] = jnp.ones((_LANES,), jnp.float32)
            return carry

        lax.fori_loop(0, C, fill_ones, 0)
        plsc.subcore_barrier()

        base = (sid * _NCORE + cid) * per_tile

        def start_idx(k, p):
            pltpu.async_copy(dst_hbm.at[pl.ds(base + k * C, C)], ID[p], SI[p])

        def wait_idx(p):
            pltpu.make_async_copy(dst_hbm.at[pl.ds(0, C)], ID[p], SI[p]).wait()

        def chunk(k, p, do_swait, next_idx):
            wait_idx(p)
            pltpu.async_copy(R[0], acc.at[ID[p]], SS[p], add=True)
            if do_swait:
                wait_scatter((p - 2) % NB)
            if next_idx is not None:
                start_idx(next_idx, (p + 2) % NB)

        start_idx(0, 0)
        start_idx(1, 1)
        head = min(2, nchunks)
        for k in range(head):
            chunk(k, k % NB, k >= 2, k + 2 if k + 2 < nchunks else None)
        steady0 = head
        nsteady = max(nchunks - 2 - steady0, 0) // NB * NB

        def body(i, carry):
            k = steady0 + i * NB
            for s in range(NB):
                chunk(k + s, (steady0 + s) % NB, True, k + s + 2)
            return carry

        lax.fori_loop(0, nsteady // NB, body, 0)
        for k in range(steady0 + nsteady, nchunks):
            chunk(k, k % NB, k >= 2, k + 2 if k + 2 < nchunks else None)
        for k in range(max(nchunks - 2, 0), nchunks):
            wait_scatter(k % NB)
        plsc.subcore_barrier()

        def wait_out(p, z):
            pltpu.make_async_copy(
                R[p], out_hbm.at[pl.ds(cid * NP + row0 + z * C, C)], SS[p]).wait()

        for z in range(nzc):
            p = z % 2
            if z >= 2:
                wait_out(p, z - 2)
            _fill_iota(ID[p], row0 + z * C, C)
            pltpu.async_copy(acc.at[ID[p]], R[p], SG[p])
            pltpu.make_async_copy(acc.at[ID[p]], R[p], SG[p]).wait()
            pltpu.async_copy(R[p], out_hbm.at[pl.ds(cid * NP + row0 + z * C, C)],
                             SS[p])
        for z in range(max(nzc - 2, 0), nzc):
            wait_out(z % 2, z)

    return deg


@functools.lru_cache(maxsize=None)
def _spmm_kernel(N, E, D):
    NP = _pad_nodes(N)
    per_tile = E // (_NCORE * _NSUB)
    assert per_tile * _NCORE * _NSUB == E
    C = _chunk_size(per_tile)
    nchunks = per_tile // C
    assert nchunks >= 3
    rows_pt = NP // _NSUB
    assert rows_pt % C == 0
    mesh = plsc.VectorSubcoreMesh(core_axis_name="c", subcore_axis_name="s")

    NB = 4

    @functools.partial(
        pl.kernel,
        out_type=jax.ShapeDtypeStruct((_NCORE * NP, D), jnp.float32),
        mesh=mesh,
        scratch_types=(
            [pltpu.VMEM((C,), jnp.int32)] * (2 * NB)
            + [pltpu.VMEM((C, D), jnp.float32)] * NB
            + [pltpu.VMEM_SHARED((NP, D), jnp.float32)]
            + [pltpu.SemaphoreType.DMA] * (3 * NB)
        ),
    )
    def spmm(src_hbm, dst_hbm, tbl_hbm, out_hbm, *refs):
        IS = refs[0:NB]
        ID = refs[NB:2 * NB]
        R = refs[2 * NB:3 * NB]
        acc = refs[3 * NB]
        SG = refs[3 * NB + 1:3 * NB + 1 + NB]
        SS = refs[3 * NB + 1 + NB:3 * NB + 1 + 2 * NB]
        SI = refs[3 * NB + 1 + 2 * NB:3 * NB + 1 + 3 * NB]
        cid = lax.axis_index("c")
        sid = lax.axis_index("s")
        row0 = sid * rows_pt

        def wait_scatter(p):
            pltpu.make_async_copy(R[p], acc.at[ID[p]], SS[p]).wait()

        _zero_rows(R[0], C, D)
        nzc = rows_pt // C
        for z in range(nzc):
            if z >= NB:
                wait_scatter(z % NB)
            _fill_iota(ID[z % NB], row0 + z * C, C)
            pltpu.async_copy(R[0], acc.at[ID[z % NB]], SS[z % NB])
        for z in range(max(nzc - NB, 0), nzc):
            wait_scatter(z % NB)
        plsc.subcore_barrier()

        ebase = (sid * _NCORE + cid) * per_tile

        def start_idx(k, p):
            off = ebase + k * C
            pltpu.async_copy(src_hbm.at[pl.ds(off, C)], IS[p], SI[p])
            pltpu.async_copy(dst_hbm.at[pl.ds(off, C)], ID[p], SI[p])

        def wait_idx(p):
            pltpu.make_async_copy(src_hbm.at[pl.ds(0, C)], IS[p], SI[p]).wait()
            pltpu.make_async_copy(dst_hbm.at[pl.ds(0, C)], ID[p], SI[p]).wait()

        def start_gather(p):
            pltpu.async_copy(tbl_hbm.at[IS[p]], R[p], SG[p])

        def wait_gather(p):
            pltpu.make_async_copy(tbl_hbm.at[IS[p]], R[p], SG[p]).wait()

        def chunk(k, p, do_swait, do_next_gather, next_idx):
            wait_gather(p)
            pltpu.async_copy(R[p], acc.at[ID[p]], SS[p], add=True)
            if do_swait:
                wait_scatter((p - 2) % NB)
            if do_next_gather:
                q = (p + 1) % NB
                wait_idx(q)
                start_gather(q)
            if next_idx is not None:
                start_idx(next_idx, (p + 2) % NB)

        start_idx(0, 0)
        start_idx(1, 1)
        wait_idx(0)
        start_gather(0)

        head = min(2, nchunks)
        for k in range(head):
            chunk(k, k % NB, k >= 2, k + 1 < nchunks,
                  k + 2 if k + 2 < nchunks else None)
        steady0 = head
        nsteady = max(nchunks - 2 - steady0, 0) // NB * NB

        def body(i, carry):
            k = steady0 + i * NB
            for s in range(NB):
                chunk(k + s, (steady0 + s) % NB, True, True, k + s + 2)
            return carry

        lax.fori_loop(0, nsteady // NB, body, 0)
        for k in range(steady0 + nsteady, nchunks):
            chunk(k, k % NB, k >= 2, k + 1 < nchunks,
                  k + 2 if k + 2 < nchunks else None)
        for k in range(max(nchunks - 2, 0), nchunks):
            wait_scatter(k % NB)
        plsc.subcore_barrier()

        def wait_out(p, z):
            pltpu.make_async_copy(
                R[p], out_hbm.at[pl.ds(cid * NP + row0 + z * C, C)], SS[p]).wait()

        for z in range(nzc):
            p = z % 2
            if z >= 2:
                wait_out(p, z - 2)
            _fill_iota(IS[p], row0 + z * C, C)
            pltpu.async_copy(acc.at[IS[p]], R[p], SG[p])
            pltpu.make_async_copy(acc.at[IS[p]], R[p], SG[p]).wait()
            pltpu.async_copy(R[p], out_hbm.at[pl.ds(cid * NP + row0 + z * C, C)],
                             SS[p])
        for z in range(max(nzc - 2, 0), nzc):
            wait_out(z % 2, z)

    return spmm


def _bn(z, g, be):
    mu = jnp.mean(z, axis=0, keepdims=True)
    var = jnp.mean((z - mu) ** 2, axis=0, keepdims=True)
    return (z - mu) * lax.rsqrt(var + 1e-5) * g + be


def _prelu(z, a):
    return jnp.where(z >= 0.0, z, a * z)


@functools.lru_cache(maxsize=None)
def _pre_kernel(N, D):
    NP = _pad_nodes(N)

    def body(degp, x, xs, dinv):
        deg = degp[0:N, 0:1] + degp[NP:NP + N, 0:1] + 1.0
        dv = lax.rsqrt(deg)
        dinv[...] = dv
        xs[0:N, :] = x[...] * dv
        if NP > N:
            xs[N:NP, :] = jnp.zeros((NP - N, D), jnp.float32)

    return pl.pallas_call(
        body,
        out_shape=(
            jax.ShapeDtypeStruct((NP, D), jnp.float32),
            jax.ShapeDtypeStruct((N, 1), jnp.float32),
        ),
    )


@functools.lru_cache(maxsize=None)
def _mid_kernel(N, D):
    NP = _pad_nodes(N)

    def body(sx, xs, dinv, bv, gv, bev, av, w1, w2, out):
        dv = dinv[...]
        st = sx[0:N, :] + sx[NP:NP + N, :] + xs[0:N, :]
        conv = jnp.dot(st, w1[...], preferred_element_type=jnp.float32) * dv + bv[...]
        z = _prelu(_bn(conv, gv[...], bev[...]), av[0, 0])
        out[0:N, :] = jnp.dot(z, w2[...], preferred_element_type=jnp.float32) * dv
        if NP > N:
            out[N:NP, :] = jnp.zeros((NP - N, D), jnp.float32)

    return pl.pallas_call(
        body, out_shape=jax.ShapeDtypeStruct((NP, D), jnp.float32))


@functools.lru_cache(maxsize=None)
def _last_kernel(N, D):
    NP = _pad_nodes(N)

    def body(p, hs, dinv, bv, gv, bev, av, out):
        conv = (p[0:N, :] + p[NP:NP + N, :] + hs[0:N, :]) * dinv[...] + bv[...]
        out[...] = _prelu(_bn(conv, gv[...], bev[...]), av[0, 0])

    return pl.pallas_call(
        body, out_shape=jax.ShapeDtypeStruct((N, D), jnp.float32))


@functools.lru_cache(maxsize=None)
def _mix_kernel(N, D):
    def body(za, zb, mixw, out):
        m = jax.nn.softmax(mixw[...], axis=-1)
        out[...] = m[:, 0:1] * za[...] + m[:, 1:2] * zb[...]

    return pl.pallas_call(
        body, out_shape=jax.ShapeDtypeStruct((N, D), jnp.float32))


def kernel(x, edge_index, W, b, gamma, beta, mix_weight, prelu_a):
    N, D = x.shape
    E = edge_index.shape[1]
    src = edge_index[0].astype(jnp.int32)
    dst = edge_index[1].astype(jnp.int32)
    row = lambda v: v.reshape(1, -1)
    pa2 = prelu_a.reshape(1, 1)

    degp = _deg_kernel(N, E, D)(dst)
    xs, dinv = _pre_kernel(N, D)(degp, x)
    sx = _spmm_kernel(N, E, D)(src, dst, xs)
    hs_a = _mid_kernel(N, D)(sx, xs, dinv, row(b[0]), row(gamma[0]),
                             row(beta[0]), pa2, W[0], W[1])
    hs_b = _mid_kernel(N, D)(sx, xs, dinv, row(b[2]), row(gamma[2]),
                             row(beta[2]), pa2, W[2], W[3])
    p_a = _spmm_kernel(N, E, D)(src, dst, hs_a)
    p_b = _spmm_kernel(N, E, D)(src, dst, hs_b)
    za = _last_kernel(N, D)(p_a, hs_a, dinv, row(b[1]), row(gamma[1]),
                            row(beta[1]), pa2)
    zb = _last_kernel(N, D)(p_b, hs_b, dinv, row(b[3]), row(gamma[3]),
                            row(beta[3]), pa2)
    return _mix_kernel(N, D)(za, zb, mix_weight)


# speedup vs baseline: 22.1890x; 1.0362x over previous
import functools

import jax
import jax.numpy as jnp
from jax import lax
from jax.experimental import pallas as pl
from jax.experimental.pallas import tpu as pltpu
from jax.experimental.pallas import tpu_sc as plsc

_NCORE = 2
_NSUB = 16
_LANES = 16


def _chunk_size(per_tile):
    for c in (128, 112, 96, 80, 64, 48, 32, 16):
        if per_tile % c == 0:
            return c
    raise ValueError(f"no chunk size for per-tile edge count {per_tile}")


def _pad_nodes(N):
    return ((N + _NSUB * _LANES - 1) // (_NSUB * _LANES)) * (_NSUB * _LANES)


def _fill_iota(idx_ref, start, cnt):
    assert cnt % _LANES == 0
    for j in range(cnt // _LANES):
        idx_ref[pl.ds(j * _LANES, _LANES)] = (
            lax.iota(jnp.int32, _LANES) + (start + j * _LANES))


def _zero_rows(rows_ref, cnt, D):
    def body(i, carry):
        for j in range(D // _LANES):
            rows_ref[i, pl.ds(j * _LANES, _LANES)] = jnp.zeros((_LANES,), jnp.float32)
        return carry

    lax.fori_loop(0, cnt, body, 0)


@functools.lru_cache(maxsize=None)
def _deg_kernel(N, E, D):
    NP = _pad_nodes(N)
    per_tile = E // (_NCORE * _NSUB)
    assert per_tile * _NCORE * _NSUB == E
    C = _chunk_size(per_tile)
    nchunks = per_tile // C
    rows_pt = NP // _NSUB
    assert rows_pt % C == 0
    mesh = plsc.VectorSubcoreMesh(core_axis_name="c", subcore_axis_name="s")

    NB = 4

    @functools.partial(
        pl.kernel,
        out_type=jax.ShapeDtypeStruct((_NCORE * NP, D), jnp.float32),
        mesh=mesh,
        scratch_types=(
            [pltpu.VMEM((C,), jnp.int32)] * NB
            + [pltpu.VMEM((C, D), jnp.float32)] * 2
            + [pltpu.VMEM_SHARED((NP, D), jnp.float32)]
            + [pltpu.SemaphoreType.DMA] * (2 * NB + 2)
        ),
    )
    def deg(dst_hbm, out_hbm, *refs):
        ID = refs[0:NB]
        R = refs[NB:NB + 2]
        acc = refs[NB + 2]
        SS = refs[NB + 3:NB + 3 + NB]
        SI = refs[NB + 3 + NB:NB + 3 + 2 * NB]
        SG = refs[NB + 3 + 2 * NB:NB + 3 + 2 * NB + 2]
        cid = lax.axis_index("c")
        sid = lax.axis_index("s")
        row0 = sid * rows_pt

        def wait_scatter(p):
            pltpu.make_async_copy(R[0], acc.at[ID[p]], SS[p]).wait()

        _zero_rows(R[0], C, D)
        nzc = rows_pt // C
        for z in range(nzc):
            if z >= NB:
                wait_scatter(z % NB)
            _fill_iota(ID[z % NB], row0 + z * C, C)
            pltpu.async_copy(R[0], acc.at[ID[z % NB]], SS[z % NB])
        for z in range(max(nzc - NB, 0), nzc):
            wait_scatter(z % NB)

        def fill_ones(i, carry):
            for j in range(D // _LANES):
                R[0][i, pl.ds(j * _LANES, _LANES)] = jnp.ones((_LANES,), jnp.float32)
            return carry

        lax.fori_loop(0, C, fill_ones, 0)
        plsc.subcore_barrier()

        base = (sid * _NCORE + cid) * per_tile

        def start_idx(k, p):
            pltpu.async_copy(dst_hbm.at[pl.ds(base + k * C, C)], ID[p], SI[p])

        def wait_idx(p):
            pltpu.make_async_copy(dst_hbm.at[pl.ds(0, C)], ID[p], SI[p]).wait()

        def chunk(k, p, do_swait, next_idx):
            wait_idx(p)
            pltpu.async_copy(R[0], acc.at[ID[p]], SS[p], add=True)
            if do_swait:
                wait_scatter((p - 2) % NB)
            if next_idx is not None:
                start_idx(next_idx, (p + 2) % NB)

        start_idx(0, 0)
        start_idx(1, 1)
        head = min(2, nchunks)
        for k in range(head):
            chunk(k, k % NB, k >= 2, k + 2 if k + 2 < nchunks else None)
        steady0 = head
        nsteady = max(nchunks - 2 - steady0, 0) // NB * NB

        def body(i, carry):
            k = steady0 + i * NB
            for s in range(NB):
                chunk(k + s, (steady0 + s) % NB, True, k + s + 2)
            return carry

        lax.fori_loop(0, nsteady // NB, body, 0)
        for k in range(steady0 + nsteady, nchunks):
            chunk(k, k % NB, k >= 2, k + 2 if k + 2 < nchunks else None)
        for k in range(max(nchunks - 2, 0), nchunks):
            wait_scatter(k % NB)
        plsc.subcore_barrier()

        def wait_out(p, z):
            pltpu.make_async_copy(
                R[p], out_hbm.at[pl.ds(cid * NP + row0 + z * C, C)], SS[p]).wait()

        for z in range(nzc):
            p = z % 2
            if z >= 2:
                wait_out(p, z - 2)
            _fill_iota(ID[p], row0 + z * C, C)
            pltpu.async_copy(acc.at[ID[p]], R[p], SG[p])
            pltpu.make_async_copy(acc.at[ID[p]], R[p], SG[p]).wait()
            pltpu.async_copy(R[p], out_hbm.at[pl.ds(cid * NP + row0 + z * C, C)],
                             SS[p])
        for z in range(max(nzc - 2, 0), nzc):
            wait_out(z % 2, z)

    return deg


@functools.lru_cache(maxsize=None)
def _spmm_kernel(N, E, D, dual=False):
    NP = _pad_nodes(N)
    nsplit = _NSUB if dual else (_NCORE * _NSUB)
    per_tile = E // nsplit
    assert per_tile * nsplit == E
    C = _chunk_size(per_tile)
    nchunks = per_tile // C
    assert nchunks >= 3
    rows_pt = NP // _NSUB
    assert rows_pt % C == 0
    mesh = plsc.VectorSubcoreMesh(core_axis_name="c", subcore_axis_name="s")

    NB = 4

    @functools.partial(
        pl.kernel,
        out_type=jax.ShapeDtypeStruct((_NCORE * NP, D), jnp.float32),
        mesh=mesh,
        scratch_types=(
            [pltpu.VMEM((C,), jnp.int32)] * (2 * NB)
            + [pltpu.VMEM((C, D), jnp.float32)] * NB
            + [pltpu.VMEM_SHARED((NP, D), jnp.float32)]
            + [pltpu.SemaphoreType.DMA] * (3 * NB)
        ),
    )
    def spmm(src_hbm, dst_hbm, tbl_hbm, out_hbm, *refs):
        IS = refs[0:NB]
        ID = refs[NB:2 * NB]
        R = refs[2 * NB:3 * NB]
        acc = refs[3 * NB]
        SG = refs[3 * NB + 1:3 * NB + 1 + NB]
        SS = refs[3 * NB + 1 + NB:3 * NB + 1 + 2 * NB]
        SI = refs[3 * NB + 1 + 2 * NB:3 * NB + 1 + 3 * NB]
        cid = lax.axis_index("c")
        sid = lax.axis_index("s")
        row0 = sid * rows_pt

        def wait_scatter(p):
            pltpu.make_async_copy(R[p], acc.at[ID[p]], SS[p]).wait()

        _zero_rows(R[0], C, D)
        nzc = rows_pt // C
        for z in range(nzc):
            if z >= NB:
                wait_scatter(z % NB)
            _fill_iota(ID[z % NB], row0 + z * C, C)
            pltpu.async_copy(R[0], acc.at[ID[z % NB]], SS[z % NB])
        for z in range(max(nzc - NB, 0), nzc):
            wait_scatter(z % NB)
        plsc.subcore_barrier()

        ebase = (sid if dual else sid * _NCORE + cid) * per_tile
        tbl0 = cid * NP

        def start_idx(k, p):
            off = ebase + k * C
            pltpu.async_copy(src_hbm.at[pl.ds(off, C)], IS[p], SI[p])
            pltpu.async_copy(dst_hbm.at[pl.ds(off, C)], ID[p], SI[p])

        def wait_idx(p):
            pltpu.make_async_copy(src_hbm.at[pl.ds(0, C)], IS[p], SI[p]).wait()
            pltpu.make_async_copy(dst_hbm.at[pl.ds(0, C)], ID[p], SI[p]).wait()
            if dual:
                for j in range(C // _LANES):
                    sl = pl.ds(j * _LANES, _LANES)
                    IS[p][sl] = IS[p][sl] + tbl0

        def start_gather(p):
            pltpu.async_copy(tbl_hbm.at[IS[p]], R[p], SG[p])

        def wait_gather(p):
            pltpu.make_async_copy(tbl_hbm.at[IS[p]], R[p], SG[p]).wait()

        def chunk(k, p, do_swait, do_next_gather, next_idx):
            wait_gather(p)
            pltpu.async_copy(R[p], acc.at[ID[p]], SS[p], add=True)
            if do_swait:
                wait_scatter((p - 2) % NB)
            if do_next_gather:
                q = (p + 1) % NB
                wait_idx(q)
                start_gather(q)
            if next_idx is not None:
                start_idx(next_idx, (p + 2) % NB)

        start_idx(0, 0)
        start_idx(1, 1)
        wait_idx(0)
        start_gather(0)

        head = min(2, nchunks)
        for k in range(head):
            chunk(k, k % NB, k >= 2, k + 1 < nchunks,
                  k + 2 if k + 2 < nchunks else None)
        steady0 = head
        nsteady = max(nchunks - 2 - steady0, 0) // NB * NB

        def body(i, carry):
            k = steady0 + i * NB
            for s in range(NB):
                chunk(k + s, (steady0 + s) % NB, True, True, k + s + 2)
            return carry

        lax.fori_loop(0, nsteady // NB, body, 0)
        for k in range(steady0 + nsteady, nchunks):
            chunk(k, k % NB, k >= 2, k + 1 < nchunks,
                  k + 2 if k + 2 < nchunks else None)
        for k in range(max(nchunks - 2, 0), nchunks):
            wait_scatter(k % NB)
        plsc.subcore_barrier()

        def wait_out(p, z):
            pltpu.make_async_copy(
                R[p], out_hbm.at[pl.ds(cid * NP + row0 + z * C, C)], SS[p]).wait()

        for z in range(nzc):
            p = z % 2
            if z >= 2:
                wait_out(p, z - 2)
            _fill_iota(IS[p], row0 + z * C, C)
            pltpu.async_copy(acc.at[IS[p]], R[p], SG[p])
            pltpu.make_async_copy(acc.at[IS[p]], R[p], SG[p]).wait()
            pltpu.async_copy(R[p], out_hbm.at[pl.ds(cid * NP + row0 + z * C, C)],
                             SS[p])
        for z in range(max(nzc - 2, 0), nzc):
            wait_out(z % 2, z)

    return spmm


def _bn(z, g, be):
    mu = jnp.mean(z, axis=0, keepdims=True)
    var = jnp.mean((z - mu) ** 2, axis=0, keepdims=True)
    return (z - mu) * lax.rsqrt(var + 1e-5) * g + be


def _prelu(z, a):
    return jnp.where(z >= 0.0, z, a * z)


@functools.lru_cache(maxsize=None)
def _pre_kernel(N, D):
    NP = _pad_nodes(N)

    def body(degp, x, xs, dinv):
        deg = degp[0:N, 0:1] + degp[NP:NP + N, 0:1] + 1.0
        dv = lax.rsqrt(deg)
        dinv[...] = dv
        xs[0:N, :] = x[...] * dv
        if NP > N:
            xs[N:NP, :] = jnp.zeros((NP - N, D), jnp.float32)

    return pl.pallas_call(
        body,
        out_shape=(
            jax.ShapeDtypeStruct((NP, D), jnp.float32),
            jax.ShapeDtypeStruct((N, 1), jnp.float32),
        ),
    )


@functools.lru_cache(maxsize=None)
def _mid_kernel(N, D):
    NP = _pad_nodes(N)

    def body(sx, xs, dinv, b_a, g_a, be_a, b_b, g_b, be_b, av, w0, w1, w2, w3,
             out):
        dv = dinv[...]
        a = av[0, 0]
        st = sx[0:N, :] + sx[NP:NP + N, :] + xs[0:N, :]
        zpad = jnp.zeros((NP - N, D), jnp.float32) if NP > N else None
        conv = jnp.dot(st, w0[...], preferred_element_type=jnp.float32) * dv + b_a[...]
        z = _prelu(_bn(conv, g_a[...], be_a[...]), a)
        out[0:N, :] = jnp.dot(z, w1[...], preferred_element_type=jnp.float32) * dv
        if NP > N:
            out[N:NP, :] = zpad
        conv = jnp.dot(st, w2[...], preferred_element_type=jnp.float32) * dv + b_b[...]
        z = _prelu(_bn(conv, g_b[...], be_b[...]), a)
        out[NP:NP + N, :] = jnp.dot(z, w3[...], preferred_element_type=jnp.float32) * dv
        if NP > N:
            out[NP + N:2 * NP, :] = zpad

    return pl.pallas_call(
        body, out_shape=jax.ShapeDtypeStruct((2 * NP, D), jnp.float32))


@functools.lru_cache(maxsize=None)
def _last_kernel(N, D):
    NP = _pad_nodes(N)

    def body(p2, hs2, dinv, b_a, g_a, be_a, b_b, g_b, be_b, av, mixw, out):
        dv = dinv[...]
        a = av[0, 0]
        m = jax.nn.softmax(mixw[...], axis=-1)
        conv = (p2[0:N, :] + hs2[0:N, :]) * dv + b_a[...]
        out[...] = m[:, 0:1] * _prelu(_bn(conv, g_a[...], be_a[...]), a)
        conv = (p2[NP:NP + N, :] + hs2[NP:NP + N, :]) * dv + b_b[...]
        out[...] = out[...] + m[:, 1:2] * _prelu(_bn(conv, g_b[...], be_b[...]), a)

    return pl.pallas_call(
        body, out_shape=jax.ShapeDtypeStruct((N, D), jnp.float32),
        compiler_params=pltpu.CompilerParams(vmem_limit_bytes=62 * 1024 * 1024))


def kernel(x, edge_index, W, b, gamma, beta, mix_weight, prelu_a):
    N, D = x.shape
    E = edge_index.shape[1]
    src = edge_index[0].astype(jnp.int32)
    dst = edge_index[1].astype(jnp.int32)
    row = lambda v: v.reshape(1, -1)
    pa2 = prelu_a.reshape(1, 1)

    degp = _deg_kernel(N, E, D)(dst)
    xs, dinv = _pre_kernel(N, D)(degp, x)
    sx = _spmm_kernel(N, E, D)(src, dst, xs)
    hs2 = _mid_kernel(N, D)(sx, xs, dinv,
                            row(b[0]), row(gamma[0]), row(beta[0]),
                            row(b[2]), row(gamma[2]), row(beta[2]),
                            pa2, W[0], W[1], W[2], W[3])
    p2 = _spmm_kernel(N, E, D, dual=True)(src, dst, hs2)
    return _last_kernel(N, D)(p2, hs2, dinv,
                              row(b[1]), row(gamma[1]), row(beta[1]),
                              row(b[3]), row(gamma[3]), row(beta[3]),
                              pa2, mix_weight)
